# baseline (speedup 1.0000x reference)
import functools

import jax
import jax.numpy as jnp
from jax import lax
from jax.experimental import pallas as pl
from jax.experimental.pallas import tpu as pltpu
from jax.experimental.pallas import tpu_sc as plsc

_N = 10000
_E = 320000
_D_IN = 128
_D_HID = 128
_D_OUT = 32
_D_CAT = 64

_NC = 2
_NS = 16
_NW = _NC * _NS
_CHUNK = 128
_CH_PW = 80
_EPW = _CHUNK * _CH_PW
_E_PAD = _EPW * _NW
_ROWS = 10240
_ROWS_PT = _ROWS // _NS
_DEG_W = 16

_BR = 1000
_GRID = _N // _BR


def _sc_mesh():
    return plsc.VectorSubcoreMesh(core_axis_name="c", subcore_axis_name="s")


def _make_deg_kernel():
    @functools.partial(
        pl.kernel,
        out_type=jax.ShapeDtypeStruct((_NC, _ROWS, _DEG_W), jnp.float32),
        mesh=_sc_mesh(),
        compiler_params=pltpu.CompilerParams(use_tc_tiling_on_sc=False),
        scratch_types=[
            pltpu.VMEM((_CH_PW, _CHUNK), jnp.int32),
            pltpu.VMEM((_CHUNK, _DEG_W), jnp.float32),
            pltpu.VMEM_SHARED((_ROWS, _DEG_W), jnp.float32),
        ],
    )
    def deg_kernel(dst_hbm, ones_hbm, zeros_hbm, out_hbm, dst_v, ones_v, acc):
        c = lax.axis_index("c")
        s = lax.axis_index("s")
        wid = c * _NS + s
        pltpu.sync_copy(zeros_hbm, acc.at[pl.ds(s * _ROWS_PT, _ROWS_PT)])
        pltpu.sync_copy(dst_hbm.at[pl.ds(wid * _CH_PW, _CH_PW)], dst_v)
        pltpu.sync_copy(ones_hbm, ones_v)
        plsc.subcore_barrier()

        def body(g, carry):
            pltpu.sync_copy(ones_v, acc.at[dst_v.at[g]], add=True)
            return carry

        lax.fori_loop(0, _CH_PW, body, 0)
        plsc.subcore_barrier()
        pltpu.sync_copy(
            acc.at[pl.ds(s * _ROWS_PT, _ROWS_PT)],
            out_hbm.at[c].at[pl.ds(s * _ROWS_PT, _ROWS_PT)],
        )

    return deg_kernel


def _make_segsum_kernel(d):
    nbuf = 2
    win = 16
    ch_fast = 144
    ch_slow = 16
    fast_core = 0

    sub = 4

    @functools.partial(
        pl.kernel,
        out_type=jax.ShapeDtypeStruct((_NC, _ROWS, d), jnp.float32),
        mesh=_sc_mesh(),
        compiler_params=pltpu.CompilerParams(use_tc_tiling_on_sc=False),
        scratch_types=[
            pltpu.VMEM((win, _CHUNK), jnp.int32),
            pltpu.VMEM((win, _CHUNK), jnp.int32),
        ] + [pltpu.VMEM((_CHUNK, d), jnp.float32)] * nbuf
          + [pltpu.VMEM_SHARED((_ROWS, d), jnp.float32)]
          + [pltpu.SemaphoreType.DMA] * (nbuf * sub),
    )
    def segsum_kernel(x_hbm, src_hbm, dst_hbm, zeros_hbm, out_hbm,
                      src_v, dst_v, *bufs_sems):
        bufs = bufs_sems[:nbuf]
        acc = bufs_sems[nbuf]
        gsems = bufs_sems[nbuf + 1:]
        c = lax.axis_index("c")
        s = lax.axis_index("s")
        is_fast = c == fast_core
        pltpu.sync_copy(zeros_hbm, acc.at[pl.ds(s * _ROWS_PT, _ROWS_PT)])
        plsc.subcore_barrier()

        @pl.when(is_fast)
        def _fast():
            def g_start(j):
                k = j % nbuf
                pltpu.async_copy(x_hbm.at[src_v.at[j]], bufs[k], gsems[k])

            def g_wait(j):
                k = j % nbuf
                pltpu.make_async_copy(x_hbm.at[src_v.at[j]], bufs[k],
                                      gsems[k]).wait()

            def body(u, carry):
                row0 = s * ch_fast + u * win
                pltpu.sync_copy(src_hbm.at[pl.ds(row0, win)], src_v)
                pltpu.sync_copy(dst_hbm.at[pl.ds(row0, win)], dst_v)
                for j in range(nbuf):
                    g_start(j)
                for j in range(win):
                    g_wait(j)
                    pltpu.sync_copy(bufs[j % nbuf], acc.at[dst_v.at[j]],
                                    add=True)
                    if j + nbuf < win:
                        g_start(j + nbuf)
                return carry

            lax.fori_loop(0, ch_fast // win, body, 0)

        @pl.when(jnp.logical_not(is_fast))
        def _slow():
            row0 = _NS * ch_fast + s * ch_slow

            w = _CHUNK // sub

            def g_start4(g, j4):
                k = g % nbuf
                pltpu.async_copy(
                    x_hbm.at[src_v.at[g, pl.ds(w * j4, w)]],
                    bufs[k].at[pl.ds(w * j4, w)],
                    gsems[k * sub + j4])

            def g_wait4(g, j4):
                k = g % nbuf
                pltpu.make_async_copy(
                    x_hbm.at[src_v.at[g, pl.ds(w * j4, w)]],
                    bufs[k].at[pl.ds(w * j4, w)],
                    gsems[k * sub + j4]).wait()

            pltpu.sync_copy(src_hbm.at[pl.ds(row0, ch_slow)], src_v)
            pltpu.sync_copy(dst_hbm.at[pl.ds(row0, ch_slow)], dst_v)
            for g in range(nbuf):
                for j4 in range(sub):
                    g_start4(g, j4)
            for g in range(ch_slow):
                for j4 in range(sub):
                    g_wait4(g, j4)
                pltpu.sync_copy(bufs[g % nbuf], acc.at[dst_v.at[g]], add=True)
                if g + nbuf < ch_slow:
                    for j4 in range(sub):
                        g_start4(g + nbuf, j4)

        plsc.subcore_barrier()
        pltpu.sync_copy(
            acc.at[pl.ds(s * _ROWS_PT, _ROWS_PT)],
            out_hbm.at[c].at[pl.ds(s * _ROWS_PT, _ROWS_PT)],
        )

    return segsum_kernel


def _dinv(d0_ref, d1_ref):
    deg = d0_ref[0, :, 0:1] + d1_ref[0, :, 0:1] + 1.0
    return lax.rsqrt(deg)


def _matmul_kernel(x_ref, w_ref, o_ref):
    o_ref[...] = jnp.dot(x_ref[...], w_ref[...],
                         preferred_element_type=jnp.float32)


def _scale_in_kernel(xw_ref, d0_ref, d1_ref, o_ref):
    o_ref[...] = xw_ref[...] * _dinv(d0_ref, d1_ref)


def _mid_kernel(p0_ref, p1_ref, xs_ref, b1_ref, w_ref, d0_ref, d1_ref, o_ref):
    dinv = _dinv(d0_ref, d1_ref)
    h = dinv * (p0_ref[0] + p1_ref[0] + xs_ref[...]) + b1_ref[...]
    h = jnp.maximum(h, 0.0)
    o_ref[...] = jnp.dot(h, w_ref[...],
                         preferred_element_type=jnp.float32) * dinv


def _final_kernel(q0_ref, q1_ref, hs_ref, b_ref, d0_ref, d1_ref,
                  mu_ref, lv_ref):
    dinv = _dinv(d0_ref, d1_ref)
    res = dinv * (q0_ref[0] + q1_ref[0] + hs_ref[...]) + b_ref[...]
    mu_ref[...] = res[:, :_D_OUT]
    lv_ref[...] = res[:, _D_OUT:]


def _row_spec(d):
    return pl.BlockSpec((_BR, d), lambda i: (i, 0))


def _plane_spec(p, d):
    return pl.BlockSpec((1, _BR, d), lambda i, p=p: (p, i, 0))


def _full_spec(r, c):
    return pl.BlockSpec((r, c), lambda i: (0, 0))


_deg_call = _make_deg_kernel()
_seg128_call = _make_segsum_kernel(_D_HID)
_seg64_call = _make_segsum_kernel(_D_CAT)


def kernel(x, edge_index, W1, b1, W_mu, b_mu, W_lv, b_lv):
    src = edge_index[0]
    dst = edge_index[1]
    pad = _E_PAD - _E
    src_p = jnp.concatenate([src, jnp.zeros((pad,), jnp.int32)])
    trash = _N + (jnp.arange(pad, dtype=jnp.int32) % (_ROWS - _N))
    dst_p = jnp.concatenate([dst, trash])
    src2d = src_p.reshape(_NW * _CH_PW, _CHUNK)
    dst2d = dst_p.reshape(_NW * _CH_PW, _CHUNK)

    ones_blk = jnp.ones((_CHUNK, _DEG_W), jnp.float32)
    zeros_deg = jnp.zeros((_ROWS_PT, _DEG_W), jnp.float32)
    zeros_128 = jnp.zeros((_ROWS_PT, _D_HID), jnp.float32)
    zeros_64 = jnp.zeros((_ROWS_PT, _D_CAT), jnp.float32)

    deg = _deg_call(dst2d, ones_blk, zeros_deg)

    xw = pl.pallas_call(
        _matmul_kernel,
        grid=(_GRID,),
        in_specs=[_row_spec(_D_IN), _full_spec(_D_IN, _D_HID)],
        out_specs=_row_spec(_D_HID),
        out_shape=jax.ShapeDtypeStruct((_N, _D_HID), jnp.float32),
    )(x, W1)

    xs = pl.pallas_call(
        _scale_in_kernel,
        grid=(_GRID,),
        in_specs=[_row_spec(_D_HID),
                  _plane_spec(0, _DEG_W), _plane_spec(1, _DEG_W)],
        out_specs=_row_spec(_D_HID),
        out_shape=jax.ShapeDtypeStruct((_N, _D_HID), jnp.float32),
    )(xw, deg, deg)

    p = _seg128_call(xs, src2d, dst2d, zeros_128)

    Wcat = jnp.concatenate([W_mu, W_lv], axis=1)
    bcat = jnp.concatenate([b_mu, b_lv]).reshape(1, _D_CAT)

    hs = pl.pallas_call(
        _mid_kernel,
        grid=(_GRID,),
        in_specs=[_plane_spec(0, _D_HID), _plane_spec(1, _D_HID),
                  _row_spec(_D_HID),
                  _full_spec(1, _D_HID), _full_spec(_D_HID, _D_CAT),
                  _plane_spec(0, _DEG_W), _plane_spec(1, _DEG_W)],
        out_specs=_row_spec(_D_CAT),
        out_shape=jax.ShapeDtypeStruct((_N, _D_CAT), jnp.float32),
    )(p, p, xs, b1.reshape(1, _D_HID), Wcat, deg, deg)

    q = _seg64_call(hs, src2d, dst2d, zeros_64)

    mu, lv = pl.pallas_call(
        _final_kernel,
        grid=(_GRID,),
        in_specs=[_plane_spec(0, _D_CAT), _plane_spec(1, _D_CAT),
                  _row_spec(_D_CAT), _full_spec(1, _D_CAT),
                  _plane_spec(0, _DEG_W), _plane_spec(1, _DEG_W)],
        out_specs=[_row_spec(_D_OUT), _row_spec(_D_OUT)],
        out_shape=[jax.ShapeDtypeStruct((_N, _D_OUT), jnp.float32),
                   jax.ShapeDtypeStruct((_N, _D_OUT), jnp.float32)],
    )(q, q, hs, bcat, deg, deg)

    return (mu, lv)


# speedup vs baseline: 1.0022x; 1.0022x over previous
import functools

import jax
import jax.numpy as jnp
from jax import lax
from jax.experimental import pallas as pl
from jax.experimental.pallas import tpu as pltpu
from jax.experimental.pallas import tpu_sc as plsc

_N = 10000
_E = 320000
_D_IN = 128
_D_HID = 128
_D_OUT = 32
_D_CAT = 64

_NC = 2
_NS = 16
_NW = _NC * _NS
_CHUNK = 128
_CH_PW = 80
_EPW = _CHUNK * _CH_PW
_E_PAD = _EPW * _NW
_ROWS = 10240
_ROWS_PT = _ROWS // _NS
_DEG_W = 16

_BR = 1000
_GRID = _N // _BR


def _sc_mesh():
    return plsc.VectorSubcoreMesh(core_axis_name="c", subcore_axis_name="s")


def _make_deg_kernel():
    @functools.partial(
        pl.kernel,
        out_type=jax.ShapeDtypeStruct((_NC, _ROWS, _DEG_W), jnp.float32),
        mesh=_sc_mesh(),
        compiler_params=pltpu.CompilerParams(use_tc_tiling_on_sc=False),
        scratch_types=[
            pltpu.VMEM((_CH_PW, _CHUNK), jnp.int32),
            pltpu.VMEM((_CHUNK, _DEG_W), jnp.float32),
            pltpu.VMEM_SHARED((_ROWS, _DEG_W), jnp.float32),
        ],
    )
    def deg_kernel(dst_hbm, ones_hbm, zeros_hbm, out_hbm, dst_v, ones_v, acc):
        c = lax.axis_index("c")
        s = lax.axis_index("s")
        wid = c * _NS + s
        pltpu.sync_copy(zeros_hbm, acc.at[pl.ds(s * _ROWS_PT, _ROWS_PT)])
        pltpu.sync_copy(dst_hbm.at[pl.ds(wid * _CH_PW, _CH_PW)], dst_v)
        pltpu.sync_copy(ones_hbm, ones_v)
        plsc.subcore_barrier()

        def body(g, carry):
            pltpu.sync_copy(ones_v, acc.at[dst_v.at[g]], add=True)
            return carry

        lax.fori_loop(0, _CH_PW, body, 0)
        plsc.subcore_barrier()
        pltpu.sync_copy(
            acc.at[pl.ds(s * _ROWS_PT, _ROWS_PT)],
            out_hbm.at[c].at[pl.ds(s * _ROWS_PT, _ROWS_PT)],
        )

    return deg_kernel


def _make_segsum_kernel(d):
    nbuf = 2
    win = 16
    ch_fast = 144
    ch_slow = 16
    fast_core = 0

    @functools.partial(
        pl.kernel,
        out_type=jax.ShapeDtypeStruct((_NC, _ROWS, d), jnp.float32),
        mesh=_sc_mesh(),
        compiler_params=pltpu.CompilerParams(use_tc_tiling_on_sc=False),
        scratch_types=[
            pltpu.VMEM((win, _CHUNK), jnp.int32),
            pltpu.VMEM((win, _CHUNK), jnp.int32),
        ] + [pltpu.VMEM((_CHUNK, d), jnp.float32)] * nbuf
          + [pltpu.VMEM_SHARED((_ROWS, d), jnp.float32)]
          + [pltpu.SemaphoreType.DMA] * nbuf,
    )
    def segsum_kernel(x_hbm, src_hbm, dst_hbm, zeros_hbm, out_hbm,
                      src_v, dst_v, *bufs_sems):
        bufs = bufs_sems[:nbuf]
        acc = bufs_sems[nbuf]
        gsems = bufs_sems[nbuf + 1:]
        c = lax.axis_index("c")
        s = lax.axis_index("s")
        is_fast = c == fast_core
        base_row = jnp.where(is_fast, s * ch_fast,
                             _NS * ch_fast + s * ch_slow)
        n_out = jnp.where(is_fast, ch_fast // win, ch_slow // win)
        pltpu.sync_copy(zeros_hbm, acc.at[pl.ds(s * _ROWS_PT, _ROWS_PT)])
        plsc.subcore_barrier()

        def g_start(j):
            k = j % nbuf
            pltpu.async_copy(x_hbm.at[src_v.at[j]], bufs[k], gsems[k])

        def g_wait(j):
            k = j % nbuf
            pltpu.make_async_copy(x_hbm.at[src_v.at[j]], bufs[k],
                                  gsems[k]).wait()

        def body(u, carry):
            row0 = base_row + u * win
            pltpu.sync_copy(src_hbm.at[pl.ds(row0, win)], src_v)
            pltpu.sync_copy(dst_hbm.at[pl.ds(row0, win)], dst_v)
            for j in range(nbuf):
                g_start(j)
            for j in range(win):
                g_wait(j)
                pltpu.sync_copy(bufs[j % nbuf], acc.at[dst_v.at[j]], add=True)
                if j + nbuf < win:
                    g_start(j + nbuf)
            return carry

        lax.fori_loop(0, n_out, body, 0)
        plsc.subcore_barrier()
        pltpu.sync_copy(
            acc.at[pl.ds(s * _ROWS_PT, _ROWS_PT)],
            out_hbm.at[c].at[pl.ds(s * _ROWS_PT, _ROWS_PT)],
        )

    return segsum_kernel


def _dinv(d0_ref, d1_ref):
    deg = d0_ref[0, :, 0:1] + d1_ref[0, :, 0:1] + 1.0
    return lax.rsqrt(deg)


def _matmul_kernel(x_ref, w_ref, o_ref):
    o_ref[...] = jnp.dot(x_ref[...], w_ref[...],
                         preferred_element_type=jnp.float32)


def _scale_in_kernel(xw_ref, d0_ref, d1_ref, o_ref):
    o_ref[...] = xw_ref[...] * _dinv(d0_ref, d1_ref)


def _mid_kernel(p0_ref, p1_ref, xs_ref, b1_ref, w_ref, d0_ref, d1_ref, o_ref):
    dinv = _dinv(d0_ref, d1_ref)
    h = dinv * (p0_ref[0] + p1_ref[0] + xs_ref[...]) + b1_ref[...]
    h = jnp.maximum(h, 0.0)
    o_ref[...] = jnp.dot(h, w_ref[...],
                         preferred_element_type=jnp.float32) * dinv


def _final_kernel(q0_ref, q1_ref, hs_ref, b_ref, d0_ref, d1_ref,
                  mu_ref, lv_ref):
    dinv = _dinv(d0_ref, d1_ref)
    res = dinv * (q0_ref[0] + q1_ref[0] + hs_ref[...]) + b_ref[...]
    mu_ref[...] = res[:, :_D_OUT]
    lv_ref[...] = res[:, _D_OUT:]


def _row_spec(d):
    return pl.BlockSpec((_BR, d), lambda i: (i, 0))


def _plane_spec(p, d):
    return pl.BlockSpec((1, _BR, d), lambda i, p=p: (p, i, 0))


def _full_spec(r, c):
    return pl.BlockSpec((r, c), lambda i: (0, 0))


_deg_call = _make_deg_kernel()
_seg128_call = _make_segsum_kernel(_D_HID)
_seg64_call = _make_segsum_kernel(_D_CAT)


def kernel(x, edge_index, W1, b1, W_mu, b_mu, W_lv, b_lv):
    src = edge_index[0]
    dst = edge_index[1]
    pad = _E_PAD - _E
    src_p = jnp.concatenate([src, jnp.zeros((pad,), jnp.int32)])
    trash = _N + (jnp.arange(pad, dtype=jnp.int32) % (_ROWS - _N))
    dst_p = jnp.concatenate([dst, trash])
    src2d = src_p.reshape(_NW * _CH_PW, _CHUNK)
    dst2d = dst_p.reshape(_NW * _CH_PW, _CHUNK)

    ones_blk = jnp.ones((_CHUNK, _DEG_W), jnp.float32)
    zeros_deg = jnp.zeros((_ROWS_PT, _DEG_W), jnp.float32)
    zeros_128 = jnp.zeros((_ROWS_PT, _D_HID), jnp.float32)
    zeros_64 = jnp.zeros((_ROWS_PT, _D_CAT), jnp.float32)

    deg = _deg_call(dst2d, ones_blk, zeros_deg)

    xw = pl.pallas_call(
        _matmul_kernel,
        grid=(_GRID,),
        in_specs=[_row_spec(_D_IN), _full_spec(_D_IN, _D_HID)],
        out_specs=_row_spec(_D_HID),
        out_shape=jax.ShapeDtypeStruct((_N, _D_HID), jnp.float32),
    )(x, W1)

    xs = pl.pallas_call(
        _scale_in_kernel,
        grid=(_GRID,),
        in_specs=[_row_spec(_D_HID),
                  _plane_spec(0, _DEG_W), _plane_spec(1, _DEG_W)],
        out_specs=_row_spec(_D_HID),
        out_shape=jax.ShapeDtypeStruct((_N, _D_HID), jnp.float32),
    )(xw, deg, deg)

    p = _seg128_call(xs, src2d, dst2d, zeros_128)

    Wcat = jnp.concatenate([W_mu, W_lv], axis=1)
    bcat = jnp.concatenate([b_mu, b_lv]).reshape(1, _D_CAT)

    hs = pl.pallas_call(
        _mid_kernel,
        grid=(_GRID,),
        in_specs=[_plane_spec(0, _D_HID), _plane_spec(1, _D_HID),
                  _row_spec(_D_HID),
                  _full_spec(1, _D_HID), _full_spec(_D_HID, _D_CAT),
                  _plane_spec(0, _DEG_W), _plane_spec(1, _DEG_W)],
        out_specs=_row_spec(_D_CAT),
        out_shape=jax.ShapeDtypeStruct((_N, _D_CAT), jnp.float32),
    )(p, p, xs, b1.reshape(1, _D_HID), Wcat, deg, deg)

    q = _seg64_call(hs, src2d, dst2d, zeros_64)

    mu, lv = pl.pallas_call(
        _final_kernel,
        grid=(_GRID,),
        in_specs=[_plane_spec(0, _D_CAT), _plane_spec(1, _D_CAT),
                  _row_spec(_D_CAT), _full_spec(1, _D_CAT),
                  _plane_spec(0, _DEG_W), _plane_spec(1, _DEG_W)],
        out_specs=[_row_spec(_D_OUT), _row_spec(_D_OUT)],
        out_shape=[jax.ShapeDtypeStruct((_N, _D_OUT), jnp.float32),
                   jax.ShapeDtypeStruct((_N, _D_OUT), jnp.float32)],
    )(q, q, hs, bcat, deg, deg)

    return (mu, lv)
